# baseline (speedup 1.0000x reference)
import functools

import jax
import jax.numpy as jnp
from jax import lax
from jax.experimental import pallas as pl
from jax.experimental.pallas import tpu as pltpu
from jax.experimental.pallas import tpu_sc as plsc

NC = 2
NS = 16
CHUNK = 80
NBUF = 2


def _sc_segment_sums(N, D2, E, DE):
    ep_tile = E // NS
    n_chunks = ep_tile // CHUNK
    rpt = N // NS

    mesh = plsc.VectorSubcoreMesh(
        core_axis_name="c", subcore_axis_name="s", num_cores=NC, num_subcores=NS
    )

    @functools.partial(
        pl.kernel,
        out_type=(
            jax.ShapeDtypeStruct((NC, N, D2), jnp.float32),
            jax.ShapeDtypeStruct((NC, N, DE), jnp.float32),
        ),
        mesh=mesh,
        compiler_params=pltpu.CompilerParams(use_tc_tiling_on_sc=False),
        scratch_types=[
            pltpu.VMEM_SHARED((N, D2), jnp.float32),
            pltpu.VMEM_SHARED((N, DE), jnp.float32),
            pltpu.VMEM((n_chunks, CHUNK), jnp.int32),
            pltpu.VMEM((n_chunks, CHUNK), jnp.int32),
            pltpu.VMEM((NBUF, CHUNK, D2), jnp.float32),
            pltpu.VMEM((NBUF, CHUNK, DE), jnp.float32),
            [pltpu.SemaphoreType.DMA] * NBUF,
            [pltpu.SemaphoreType.DMA] * NBUF,
            pltpu.SemaphoreType.DMA,
            pltpu.SemaphoreType.DMA,
        ],
    )
    def sc_kernel(feat_hbm, src_hbm, dst_hbm, ef_hbm, zf_hbm, ze_hbm,
                  hf_out, he_out, acc_f, acc_e, src_v, dst_v, rows_v, ef_v,
                  gsems, esems, sfsem, sesem):
        c = lax.axis_index("c")
        s = lax.axis_index("s")
        do_ef = (s >= NS // 2) == (c == 1)

        pltpu.sync_copy(zf_hbm, acc_f.at[pl.ds(s * rpt, rpt)])
        pltpu.sync_copy(ze_hbm, acc_e.at[pl.ds(s * rpt, rpt)])
        pltpu.sync_copy(src_hbm.at[c, s], src_v)
        pltpu.sync_copy(dst_hbm.at[s], dst_v)
        plsc.subcore_barrier()

        def issue_gathers(j, b):
            pltpu.async_copy(feat_hbm.at[src_v.at[j]], rows_v.at[b], gsems[b])

            @pl.when(do_ef)
            def _():
                base = s * ep_tile + j * CHUNK
                pltpu.async_copy(ef_hbm.at[pl.ds(base, CHUNK)], ef_v.at[b],
                                 esems[b])

        def process(j, b, prefetch):
            pltpu.make_async_copy(feat_hbm.at[pl.ds(0, CHUNK)],
                                  rows_v.at[b], gsems[b]).wait()
            df = pltpu.async_copy(rows_v.at[b], acc_f.at[dst_v.at[j]], sfsem,
                                  add=True)

            @pl.when(do_ef)
            def _():
                pltpu.make_async_copy(ef_hbm.at[pl.ds(0, CHUNK)],
                                      ef_v.at[b], esems[b]).wait()
                pltpu.async_copy(ef_v.at[b], acc_e.at[dst_v.at[j]], sesem,
                                 add=True).wait()

            df.wait()
            if prefetch:
                issue_gathers(j + NBUF, b)

        for b in range(NBUF):
            issue_gathers(b, b)

        def body(i, carry):
            j = i * NBUF
            for b in range(NBUF):
                process(j + b, b, prefetch=True)
            return carry

        n_tail = NBUF + (n_chunks % NBUF)
        lax.fori_loop(0, (n_chunks - n_tail) // NBUF, body, 0)
        for ch in range(n_chunks - n_tail, n_chunks):
            process(ch, ch % NBUF, prefetch=(ch + NBUF < n_chunks))
        plsc.subcore_barrier()

        sl = pl.ds(s * rpt, rpt)
        pltpu.sync_copy(acc_f.at[sl], hf_out.at[c, sl])
        pltpu.sync_copy(acc_e.at[sl], he_out.at[c, sl])

    return sc_kernel


def _tc_transpose_ef(E, DE, L=3200):
    def body(in_ref, out_ref):
        out_ref[...] = in_ref[...].T

    return pl.pallas_call(
        body,
        grid=(E // L,),
        in_specs=[pl.BlockSpec((DE, L), lambda i: (0, i))],
        out_specs=pl.BlockSpec((L, DE), lambda i: (i, 0)),
        out_shape=jax.ShapeDtypeStruct((E, DE), jnp.float32),
    )


def _tc_combine(N, D, DE, R=1000):
    D2 = D // 2

    def body(feat_ref, hf_ref, he_ref, w_ref, b_ref, out_ref):
        w = w_ref[...]
        acc = jnp.dot(hf_ref[0], w[:D2], preferred_element_type=jnp.float32)
        acc += jnp.dot(hf_ref[1], w[D2:D], preferred_element_type=jnp.float32)
        acc += jnp.dot(he_ref[0] + he_ref[1], w[D:],
                       preferred_element_type=jnp.float32)
        out_ref[...] = feat_ref[...] + acc + b_ref[...]

    return pl.pallas_call(
        body,
        grid=(N // R,),
        in_specs=[
            pl.BlockSpec((R, D), lambda i: (i, 0)),
            pl.BlockSpec((NC, R, D2), lambda i: (0, i, 0)),
            pl.BlockSpec((NC, R, DE), lambda i: (0, i, 0)),
            pl.BlockSpec((D + DE, D), lambda i: (0, 0)),
            pl.BlockSpec((1, D), lambda i: (0, 0)),
        ],
        out_specs=pl.BlockSpec((R, D), lambda i: (i, 0)),
        out_shape=jax.ShapeDtypeStruct((N, D), jnp.float32),
    )


def kernel(feat, edge_index, edge_feat, W, b):
    N, D = feat.shape
    E, DE = edge_feat.shape
    D2 = D // 2
    npad = -(-N // (8 * NS)) * (8 * NS)

    nch = E // (NS * CHUNK)
    src = edge_index[0].astype(jnp.int32)
    dst = edge_index[1].astype(jnp.int32).reshape(NS, nch, CHUNK)
    src2 = (jnp.stack([src * 2, src * 2 + 1])).reshape(NC, NS, nch, CHUNK)
    feat2 = feat.reshape(N * 2, D2)
    zeros_f = jnp.zeros((npad // NS, D2), jnp.float32)
    zeros_e = jnp.zeros((npad // NS, DE), jnp.float32)

    ef_rows = _tc_transpose_ef(E, DE)(edge_feat.T)
    hf, he = _sc_segment_sums(npad, D2, E, DE)(
        feat2, src2, dst, ef_rows, zeros_f, zeros_e
    )
    return _tc_combine(N, D, DE)(feat, hf, he, W, b.reshape(1, D))


# speedup vs baseline: 1.6249x; 1.6249x over previous
import functools

import jax
import jax.numpy as jnp
from jax import lax
from jax.experimental import pallas as pl
from jax.experimental.pallas import tpu as pltpu
from jax.experimental.pallas import tpu_sc as plsc

NC = 2
NS = 16
CHUNK = 80
NBUF = 2
LANES = 16


def _sc_segment_sums(N, D2, E, DE):
    ep_tile = E // NS
    n_chunks = ep_tile // CHUNK
    half = n_chunks // 2
    rpt = N // NS

    mesh = plsc.VectorSubcoreMesh(
        core_axis_name="c", subcore_axis_name="s", num_cores=NC, num_subcores=NS
    )

    @functools.partial(
        pl.kernel,
        out_type=(
            jax.ShapeDtypeStruct((NC, N, D2), jnp.float32),
            jax.ShapeDtypeStruct((NC, N, DE), jnp.float32),
        ),
        mesh=mesh,
        compiler_params=pltpu.CompilerParams(use_tc_tiling_on_sc=False,
                                             needs_layout_passes=False),
        scratch_types=[
            pltpu.VMEM_SHARED((N, D2), jnp.float32),
            pltpu.VMEM_SHARED((N, DE), jnp.float32),
            pltpu.VMEM((n_chunks, CHUNK), jnp.int32),
            pltpu.VMEM((n_chunks, CHUNK), jnp.int32),
            pltpu.VMEM((NBUF, CHUNK, D2), jnp.float32),
            pltpu.VMEM((NBUF, DE, CHUNK), jnp.float32),
            pltpu.VMEM((CHUNK, DE), jnp.float32),
            [pltpu.SemaphoreType.DMA] * NBUF,
            [pltpu.SemaphoreType.DMA] * NBUF,
            pltpu.SemaphoreType.DMA,
        ],
    )
    def sc_kernel(feat_hbm, src_hbm, dst_hbm, eft_hbm, zf_hbm, ze_hbm,
                  hf_out, he_out, acc_f, acc_e, src_v, dst_v, rows_v, eft_v,
                  ef2d, gsems, esems, sfsem):
        c = lax.axis_index("c")
        s = lax.axis_index("s")
        ids = lax.iota(jnp.int32, LANES)

        pltpu.sync_copy(zf_hbm, acc_f.at[pl.ds(s * rpt, rpt)])
        pltpu.sync_copy(ze_hbm, acc_e.at[pl.ds(s * rpt, rpt)])
        pltpu.sync_copy(src_hbm.at[c, s], src_v)
        pltpu.sync_copy(dst_hbm.at[s], dst_v)
        plsc.subcore_barrier()

        def ef_owned(j):
            return (j < half) == (c == 0)

        def issue_gathers(j, b):
            pltpu.async_copy(feat_hbm.at[src_v.at[j]], rows_v.at[b], gsems[b])

            @pl.when(ef_owned(j))
            def _():
                base = s * ep_tile + j * CHUNK
                pltpu.async_copy(eft_hbm.at[:, pl.ds(base, CHUNK)],
                                 eft_v.at[b], esems[b])

        def process(j, b, prefetch):
            pltpu.make_async_copy(feat_hbm.at[pl.ds(0, CHUNK)],
                                  rows_v.at[b], gsems[b]).wait()
            df = pltpu.async_copy(rows_v.at[b], acc_f.at[dst_v.at[j]], sfsem,
                                  add=True)

            @pl.when(ef_owned(j))
            def _():
                pltpu.make_async_copy(eft_hbm.at[:, pl.ds(0, CHUNK)],
                                      eft_v.at[b], esems[b]).wait()
                for k in range(DE):
                    for g in range(CHUNK // LANES):
                        vals = eft_v[b, k, pl.ds(g * LANES, LANES)]
                        plsc.store_scatter(
                            ef2d, [ids + (g * LANES), ids * 0 + k], vals)
                pltpu.sync_copy(ef2d, acc_e.at[dst_v.at[j]], add=True)

            df.wait()
            if prefetch:
                issue_gathers(j + NBUF, b)

        for b in range(NBUF):
            issue_gathers(b, b)

        def body(i, carry):
            j = i * NBUF
            for b in range(NBUF):
                process(j + b, b, prefetch=True)
            return carry

        n_tail = NBUF + (n_chunks % NBUF)
        lax.fori_loop(0, (n_chunks - n_tail) // NBUF, body, 0)
        for ch in range(n_chunks - n_tail, n_chunks):
            process(ch, ch % NBUF, prefetch=(ch + NBUF < n_chunks))
        plsc.subcore_barrier()

        sl = pl.ds(s * rpt, rpt)
        pltpu.sync_copy(acc_f.at[sl], hf_out.at[c, sl])
        pltpu.sync_copy(acc_e.at[sl], he_out.at[c, sl])

    return sc_kernel


def _tc_combine(N, D, DE, R=1000):
    D2 = D // 2

    def body(feat_ref, hf_ref, he_ref, w_ref, b_ref, out_ref):
        w = w_ref[...]
        acc = jnp.dot(hf_ref[0], w[:D2], preferred_element_type=jnp.float32)
        acc += jnp.dot(hf_ref[1], w[D2:D], preferred_element_type=jnp.float32)
        acc += jnp.dot(he_ref[0] + he_ref[1], w[D:],
                       preferred_element_type=jnp.float32)
        out_ref[...] = feat_ref[...] + acc + b_ref[...]

    return pl.pallas_call(
        body,
        grid=(N // R,),
        in_specs=[
            pl.BlockSpec((R, D), lambda i: (i, 0)),
            pl.BlockSpec((NC, R, D2), lambda i: (0, i, 0)),
            pl.BlockSpec((NC, R, DE), lambda i: (0, i, 0)),
            pl.BlockSpec((D + DE, D), lambda i: (0, 0)),
            pl.BlockSpec((1, D), lambda i: (0, 0)),
        ],
        out_specs=pl.BlockSpec((R, D), lambda i: (i, 0)),
        out_shape=jax.ShapeDtypeStruct((N, D), jnp.float32),
    )


def kernel(feat, edge_index, edge_feat, W, b):
    N, D = feat.shape
    E, DE = edge_feat.shape
    D2 = D // 2
    npad = -(-N // (8 * NS)) * (8 * NS)

    nch = E // (NS * CHUNK)
    src = edge_index[0].astype(jnp.int32)
    dst = edge_index[1].astype(jnp.int32).reshape(NS, nch, CHUNK)
    src2 = (jnp.stack([src * 2, src * 2 + 1])).reshape(NC, NS, nch, CHUNK)
    feat2 = feat.reshape(N * 2, D2)
    zeros_f = jnp.zeros((npad // NS, D2), jnp.float32)
    zeros_e = jnp.zeros((npad // NS, DE), jnp.float32)

    hf, he = _sc_segment_sums(npad, D2, E, DE)(
        feat2, src2, dst, edge_feat.T, zeros_f, zeros_e
    )
    return _tc_combine(N, D, DE)(feat, hf, he, W, b.reshape(1, D))


# speedup vs baseline: 2.0997x; 1.2922x over previous
import functools

import jax
import jax.numpy as jnp
from jax import lax
from jax.experimental import pallas as pl
from jax.experimental.pallas import tpu as pltpu
from jax.experimental.pallas import tpu_sc as plsc

NC = 2
NS = 16
CHUNK = 80
NBUF = 3
LANES = 16


def _sc_segment_sums(N, D2, E, DE):
    ep_tile = E // NS
    n_chunks = ep_tile // CHUNK
    half = n_chunks // 2
    rpt = N // NS

    mesh = plsc.VectorSubcoreMesh(
        core_axis_name="c", subcore_axis_name="s", num_cores=NC, num_subcores=NS
    )

    @functools.partial(
        pl.kernel,
        out_type=(
            jax.ShapeDtypeStruct((NC, N, D2), jnp.float32),
            jax.ShapeDtypeStruct((NC, N, DE), jnp.float32),
        ),
        mesh=mesh,
        compiler_params=pltpu.CompilerParams(use_tc_tiling_on_sc=False,
                                             needs_layout_passes=False),
        scratch_types=[
            pltpu.VMEM_SHARED((N, D2), jnp.float32),
            pltpu.VMEM_SHARED((N, DE), jnp.float32),
            pltpu.VMEM((n_chunks, CHUNK), jnp.int32),
            pltpu.VMEM((n_chunks, CHUNK), jnp.int32),
            pltpu.VMEM((NBUF, CHUNK, D2), jnp.float32),
            pltpu.VMEM((NBUF, DE, CHUNK), jnp.float32),
            pltpu.VMEM((CHUNK, DE), jnp.float32),
            [pltpu.SemaphoreType.DMA] * NBUF,
            [pltpu.SemaphoreType.DMA] * NBUF,
            pltpu.SemaphoreType.DMA,
        ],
    )
    def sc_kernel(feat_hbm, ei_hbm, eft_hbm, zf_hbm, ze_hbm,
                  hf_out, he_out, acc_f, acc_e, src_v, dst_v, rows_v, eft_v,
                  ef2d, gsems, esems, sfsem):
        c = lax.axis_index("c")
        s = lax.axis_index("s")
        ids = lax.iota(jnp.int32, LANES)

        pltpu.sync_copy(zf_hbm, acc_f.at[pl.ds(s * rpt, rpt)])
        pltpu.sync_copy(ze_hbm, acc_e.at[pl.ds(s * rpt, rpt)])
        pltpu.sync_copy(ei_hbm.at[0, s], src_v)
        pltpu.sync_copy(ei_hbm.at[1, s], dst_v)

        def scale_src(j, carry):
            for g in range(CHUNK // LANES):
                sl = pl.ds(g * LANES, LANES)
                src_v[j, sl] = src_v[j, sl] * 2 + c
            return carry

        lax.fori_loop(0, n_chunks, scale_src, 0)
        plsc.subcore_barrier()

        def ef_owned(j):
            return (j < half) == (c == 0)

        def issue_gathers(j, b):
            pltpu.async_copy(feat_hbm.at[src_v.at[j]], rows_v.at[b], gsems[b])

            @pl.when(ef_owned(j))
            def _():
                base = s * ep_tile + j * CHUNK
                pltpu.async_copy(eft_hbm.at[:, pl.ds(base, CHUNK)],
                                 eft_v.at[b], esems[b])

        def process(j, b, prefetch):
            pltpu.make_async_copy(feat_hbm.at[pl.ds(0, CHUNK)],
                                  rows_v.at[b], gsems[b]).wait()
            df = pltpu.async_copy(rows_v.at[b], acc_f.at[dst_v.at[j]], sfsem,
                                  add=True)

            @pl.when(ef_owned(j))
            def _():
                pltpu.make_async_copy(eft_hbm.at[:, pl.ds(0, CHUNK)],
                                      eft_v.at[b], esems[b]).wait()
                for k in range(DE):
                    for g in range(CHUNK // LANES):
                        vals = eft_v[b, k, pl.ds(g * LANES, LANES)]
                        plsc.store_scatter(
                            ef2d, [ids + (g * LANES), ids * 0 + k], vals)
                pltpu.sync_copy(ef2d, acc_e.at[dst_v.at[j]], add=True)

            df.wait()
            if prefetch:
                issue_gathers(j + NBUF, b)

        for b in range(NBUF):
            issue_gathers(b, b)

        def body(i, carry):
            j = i * NBUF
            for b in range(NBUF):
                process(j + b, b, prefetch=True)
            return carry

        n_tail = NBUF + (n_chunks % NBUF)
        lax.fori_loop(0, (n_chunks - n_tail) // NBUF, body, 0)
        for ch in range(n_chunks - n_tail, n_chunks):
            process(ch, ch % NBUF, prefetch=(ch + NBUF < n_chunks))
        plsc.subcore_barrier()

        sl = pl.ds(s * rpt, rpt)
        pltpu.sync_copy(acc_f.at[sl], hf_out.at[c, sl])
        pltpu.sync_copy(acc_e.at[sl], he_out.at[c, sl])

    return sc_kernel


def _tc_combine(N, D, DE, R=1000):
    D2 = D // 2

    def body(feat_ref, hf_ref, he_ref, w_ref, b_ref, out_ref):
        w = w_ref[...]
        acc = jnp.dot(hf_ref[0], w[:D2], preferred_element_type=jnp.float32)
        acc += jnp.dot(hf_ref[1], w[D2:D], preferred_element_type=jnp.float32)
        acc += jnp.dot(he_ref[0] + he_ref[1], w[D:],
                       preferred_element_type=jnp.float32)
        out_ref[...] = feat_ref[...] + acc + b_ref[...]

    return pl.pallas_call(
        body,
        grid=(N // R,),
        in_specs=[
            pl.BlockSpec((R, D), lambda i: (i, 0)),
            pl.BlockSpec((NC, R, D2), lambda i: (0, i, 0)),
            pl.BlockSpec((NC, R, DE), lambda i: (0, i, 0)),
            pl.BlockSpec((D + DE, D), lambda i: (0, 0)),
            pl.BlockSpec((1, D), lambda i: (0, 0)),
        ],
        out_specs=pl.BlockSpec((R, D), lambda i: (i, 0)),
        out_shape=jax.ShapeDtypeStruct((N, D), jnp.float32),
    )


def kernel(feat, edge_index, edge_feat, W, b):
    N, D = feat.shape
    E, DE = edge_feat.shape
    D2 = D // 2
    npad = -(-N // (8 * NS)) * (8 * NS)

    nch = E // (NS * CHUNK)
    ei = edge_index.astype(jnp.int32).reshape(2, NS, nch, CHUNK)
    feat2 = feat.reshape(N * 2, D2)
    zeros_f = jnp.zeros((npad // NS, D2), jnp.float32)
    zeros_e = jnp.zeros((npad // NS, DE), jnp.float32)

    hf, he = _sc_segment_sums(npad, D2, E, DE)(
        feat2, ei, edge_feat.T, zeros_f, zeros_e
    )
    return _tc_combine(N, D, DE)(feat, hf, he, W, b.reshape(1, D))
